# Initial kernel scaffold; baseline (speedup 1.0000x reference)
import functools

import jax
import jax.numpy as jnp
from jax import lax
from jax.experimental import pallas as pl
from jax.experimental.pallas import tpu as pltpu
from jax.experimental.pallas import tpu_sc as plsc

NC = 2
NS = 16
NW = NC * NS
K = 80


def _sc_degree(dst3, N):
    nch = dst3.shape[1]
    mesh = plsc.VectorSubcoreMesh(
        core_axis_name="c", subcore_axis_name="s", num_cores=NC, num_subcores=NS)
    rpt = N // NS

    @functools.partial(
        pl.kernel,
        out_type=jax.ShapeDtypeStruct((NC, N, 1), jnp.float32),
        mesh=mesh,
        scratch_types=[
            pltpu.VMEM((nch, K), jnp.int32),
            pltpu.VMEM((K, 1), jnp.float32),
            pltpu.VMEM((rpt, 1), jnp.float32),
            pltpu.VMEM_SHARED((N, 1), jnp.float32),
        ],
    )
    def k(dst_hbm, ones_hbm, zeros_hbm, out_hbm, dst_v, ones_v, zbuf, acc_sh):
        c = lax.axis_index("c")
        s = lax.axis_index("s")
        w = s * NC + c
        pltpu.sync_copy(dst_hbm.at[w], dst_v)
        pltpu.sync_copy(ones_hbm, ones_v)
        pltpu.sync_copy(zeros_hbm, zbuf)
        pltpu.sync_copy(zbuf, acc_sh.at[pl.ds(s * rpt, rpt)])
        plsc.subcore_barrier()

        def step(j, carry):
            pltpu.sync_copy(ones_v, acc_sh.at[dst_v.at[j]], add=True)
            return carry

        lax.fori_loop(0, nch, step, 0)
        plsc.subcore_barrier()
        pltpu.sync_copy(acc_sh.at[pl.ds(s * rpt, rpt)], zbuf)
        pltpu.sync_copy(zbuf, out_hbm.at[c, pl.ds(s * rpt, rpt)])

    ones = jnp.ones((K, 1), jnp.float32)
    zeros = jnp.zeros((rpt, 1), jnp.float32)
    return k(dst3, ones, zeros)


def _sc_scatter(hs, src3, dst3):
    N, D = hs.shape
    nch = src3.shape[1]
    rpt = N // NS
    zr = rpt // 5
    mesh = plsc.VectorSubcoreMesh(
        core_axis_name="c", subcore_axis_name="s", num_cores=NC, num_subcores=NS)

    @functools.partial(
        pl.kernel,
        out_type=jax.ShapeDtypeStruct((NC, N, D), jnp.float32),
        mesh=mesh,
        scratch_types=[
            pltpu.VMEM((nch, K), jnp.int32),
            pltpu.VMEM((nch, K), jnp.int32),
            pltpu.VMEM((K, D), jnp.float32),
            pltpu.VMEM((K, D), jnp.float32),
            pltpu.VMEM((zr, D), jnp.float32),
            pltpu.VMEM_SHARED((N, D), jnp.float32),
            pltpu.SemaphoreType.DMA,
            pltpu.SemaphoreType.DMA,
        ],
    )
    def k(hs_hbm, src_hbm, dst_hbm, zeros_hbm, out_hbm,
          src_v, dst_v, rows_a, rows_b, zbuf, acc_sh, sem_a, sem_b):
        c = lax.axis_index("c")
        s = lax.axis_index("s")
        w = s * NC + c
        pltpu.sync_copy(src_hbm.at[w], src_v)
        pltpu.sync_copy(dst_hbm.at[w], dst_v)
        pltpu.sync_copy(zeros_hbm, zbuf)
        for r in range(rpt // zr):
            pltpu.sync_copy(zbuf, acc_sh.at[pl.ds(s * rpt + r * zr, zr)])
        plsc.subcore_barrier()

        pltpu.async_copy(hs_hbm.at[src_v.at[0]], rows_a, sem_a)

        def step(g, carry):
            j = 2 * g
            pltpu.async_copy(hs_hbm.at[src_v.at[j + 1]], rows_b, sem_b)
            pltpu.make_async_copy(hs_hbm.at[src_v.at[j]], rows_a, sem_a).wait()
            pltpu.sync_copy(rows_a, acc_sh.at[dst_v.at[j]], add=True)
            pltpu.async_copy(hs_hbm.at[src_v.at[j + 2]], rows_a, sem_a)
            pltpu.make_async_copy(hs_hbm.at[src_v.at[j + 1]], rows_b, sem_b).wait()
            pltpu.sync_copy(rows_b, acc_sh.at[dst_v.at[j + 1]], add=True)
            return carry

        lax.fori_loop(0, (nch - 1) // 2, step, 0)
        pltpu.make_async_copy(hs_hbm.at[src_v.at[nch - 1]], rows_a, sem_a).wait()
        pltpu.sync_copy(rows_a, acc_sh.at[dst_v.at[nch - 1]], add=True)

        plsc.subcore_barrier()
        for r in range(rpt // zr):
            pltpu.sync_copy(acc_sh.at[pl.ds(s * rpt + r * zr, zr)], zbuf)
            pltpu.sync_copy(zbuf, out_hbm.at[c, pl.ds(s * rpt + r * zr, zr)])

    zeros = jnp.zeros((zr, D), jnp.float32)
    return k(hs, src3, dst3, zeros)


def _tc_first(degp, x, W):
    N, Din = x.shape
    Dout = W.shape[1]

    def body(degp_ref, x_ref, w_ref, dis_ref, hs_ref):
        deg = degp_ref[0] + degp_ref[1] + 1.0
        dis = lax.rsqrt(deg)
        dis_ref[...] = dis
        hs_ref[...] = jnp.dot(x_ref[...], w_ref[...],
                              preferred_element_type=jnp.float32) * dis

    return pl.pallas_call(
        body,
        out_shape=(jax.ShapeDtypeStruct((N, 1), jnp.float32),
                   jax.ShapeDtypeStruct((N, Dout), jnp.float32)),
    )(degp, x, W)


def _tc_mid(acc, hs_prev, dis, b, W):
    N, D = hs_prev.shape
    Dout = W.shape[1]

    def body(acc_ref, hsp_ref, dis_ref, b_ref, w_ref, out_ref):
        agg = acc_ref[0] + acc_ref[1] + hsp_ref[...]
        h = jnp.maximum(agg * dis_ref[...] + b_ref[...], 0.0)
        out_ref[...] = jnp.dot(h, w_ref[...],
                               preferred_element_type=jnp.float32) * dis_ref[...]

    return pl.pallas_call(
        body,
        out_shape=jax.ShapeDtypeStruct((N, Dout), jnp.float32),
    )(acc, hs_prev, dis, b.reshape(1, D), W)


def _tc_final(acc, hs_prev, dis, b):
    N, D = hs_prev.shape

    def body(acc_ref, hsp_ref, dis_ref, b_ref, out_ref):
        agg = acc_ref[0] + acc_ref[1] + hsp_ref[...]
        out_ref[...] = agg * dis_ref[...] + b_ref[...]

    return pl.pallas_call(
        body,
        out_shape=jax.ShapeDtypeStruct((N, D), jnp.float32),
    )(acc, hs_prev, dis, b.reshape(1, D))


def kernel(x, edge_index, W1, b1, W2, b2, W3, b3):
    N = x.shape[0]
    E = edge_index.shape[1]
    epw = E // NW
    nch = epw // K
    src3 = edge_index[0].astype(jnp.int32).reshape(NW, nch, K)
    dst3 = edge_index[1].astype(jnp.int32).reshape(NW, nch, K)

    degp = _sc_degree(dst3, N)
    dis, hs1 = _tc_first(degp, x, W1)
    acc1 = _sc_scatter(hs1, src3, dst3)
    hs2 = _tc_mid(acc1, hs1, dis, b1, W2)
    acc2 = _sc_scatter(hs2, src3, dst3)
    hs3 = _tc_mid(acc2, hs2, dis, b2, W3)
    acc3 = _sc_scatter(hs3, src3, dst3)
    return _tc_final(acc3, hs3, dis, b3)


# speedup vs baseline: 8.7037x; 8.7037x over previous
import functools

import jax
import jax.numpy as jnp
from jax import lax
from jax.experimental import pallas as pl
from jax.experimental.pallas import tpu as pltpu
from jax.experimental.pallas import tpu_sc as plsc

NC = 2
NS = 16
NW = NC * NS
K = 128


def _pad_rows(N):
    return (-(-(N // NS + 1) // 8)) * 8


def _sc_degree(dst3, N, DW=128):
    nch = dst3.shape[1]
    mesh = plsc.VectorSubcoreMesh(
        core_axis_name="c", subcore_axis_name="s", num_cores=NC, num_subcores=NS)
    rpt = _pad_rows(N)
    npad = rpt * NS
    zr = 8

    @functools.partial(
        pl.kernel,
        out_type=jax.ShapeDtypeStruct((NC, npad, DW), jnp.float32),
        mesh=mesh,
        scratch_types=[
            pltpu.VMEM((nch, K), jnp.int32),
            pltpu.VMEM((K, DW), jnp.float32),
            pltpu.VMEM((zr, DW), jnp.float32),
            pltpu.VMEM_SHARED((npad, DW), jnp.float32),
        ],
    )
    def k(dst_hbm, ones_hbm, zeros_hbm, out_hbm, dst_v, ones_v, zbuf, acc_sh):
        c = lax.axis_index("c")
        s = lax.axis_index("s")
        w = s * NC + c
        pltpu.sync_copy(dst_hbm.at[w], dst_v)
        pltpu.sync_copy(ones_hbm, ones_v)
        pltpu.sync_copy(zeros_hbm, zbuf)

        def zstep(r, carry):
            pltpu.sync_copy(zbuf, acc_sh.at[pl.ds(s * rpt + r * zr, zr)])
            return carry

        lax.fori_loop(0, rpt // zr, zstep, 0)
        plsc.subcore_barrier()

        def step(j, carry):
            pltpu.sync_copy(ones_v, acc_sh.at[dst_v.at[j]], add=True)
            return carry

        lax.fori_loop(0, nch, step, 0)
        plsc.subcore_barrier()

        def wstep(r, carry):
            pltpu.sync_copy(acc_sh.at[pl.ds(s * rpt + r * zr, zr)], zbuf)
            pltpu.sync_copy(zbuf, out_hbm.at[c, pl.ds(s * rpt + r * zr, zr)])
            return carry

        lax.fori_loop(0, rpt // zr, wstep, 0)

    ones = jnp.ones((K, DW), jnp.float32)
    zeros = jnp.zeros((zr, DW), jnp.float32)
    return k(dst3, ones, zeros)


def _sc_scatter(hs, src3, dst3):
    N, D = hs.shape
    nch = src3.shape[1]
    half = nch // 2
    rpt = _pad_rows(N)
    npad = rpt * NS
    zr = 8
    mesh = plsc.VectorSubcoreMesh(
        core_axis_name="c", subcore_axis_name="s", num_cores=NC, num_subcores=NS)

    @functools.partial(
        pl.kernel,
        out_type=jax.ShapeDtypeStruct((NC, npad, D), jnp.float32),
        mesh=mesh,
        scratch_types=[
            pltpu.VMEM((half, K), jnp.int32),
            pltpu.VMEM((half, K), jnp.int32),
            pltpu.VMEM((K, D), jnp.float32),
            pltpu.VMEM((K, D), jnp.float32),
            pltpu.VMEM((zr, D), jnp.float32),
            pltpu.VMEM_SHARED((npad, D), jnp.float32),
            pltpu.SemaphoreType.DMA,
            pltpu.SemaphoreType.DMA,
        ],
    )
    def k(hs_hbm, src_hbm, dst_hbm, zeros_hbm, out_hbm,
          src_v, dst_v, rows_a, rows_b, zbuf, acc_sh, sem_a, sem_b):
        c = lax.axis_index("c")
        s = lax.axis_index("s")
        w = s * NC + c
        pltpu.sync_copy(zeros_hbm, zbuf)

        def zstep(r, carry):
            pltpu.sync_copy(zbuf, acc_sh.at[pl.ds(s * rpt + r * zr, zr)])
            return carry

        lax.fori_loop(0, rpt // zr, zstep, 0)
        plsc.subcore_barrier()

        for h in range(2):
            pltpu.sync_copy(src_hbm.at[w, pl.ds(h * half, half)], src_v)
            pltpu.sync_copy(dst_hbm.at[w, pl.ds(h * half, half)], dst_v)
            pltpu.async_copy(hs_hbm.at[src_v.at[0]], rows_a, sem_a)

            def step(g, carry):
                j = 2 * g
                pltpu.async_copy(hs_hbm.at[src_v.at[j + 1]], rows_b, sem_b)
                pltpu.make_async_copy(
                    hs_hbm.at[src_v.at[j]], rows_a, sem_a).wait()
                pltpu.sync_copy(rows_a, acc_sh.at[dst_v.at[j]], add=True)
                pltpu.async_copy(hs_hbm.at[src_v.at[j + 2]], rows_a, sem_a)
                pltpu.make_async_copy(
                    hs_hbm.at[src_v.at[j + 1]], rows_b, sem_b).wait()
                pltpu.sync_copy(rows_b, acc_sh.at[dst_v.at[j + 1]], add=True)
                return carry

            lax.fori_loop(0, half // 2 - 1, step, 0)
            pltpu.async_copy(hs_hbm.at[src_v.at[half - 1]], rows_b, sem_b)
            pltpu.make_async_copy(
                hs_hbm.at[src_v.at[half - 2]], rows_a, sem_a).wait()
            pltpu.sync_copy(rows_a, acc_sh.at[dst_v.at[half - 2]], add=True)
            pltpu.make_async_copy(
                hs_hbm.at[src_v.at[half - 1]], rows_b, sem_b).wait()
            pltpu.sync_copy(rows_b, acc_sh.at[dst_v.at[half - 1]], add=True)

        plsc.subcore_barrier()

        def wstep(r, carry):
            pltpu.sync_copy(acc_sh.at[pl.ds(s * rpt + r * zr, zr)], zbuf)
            pltpu.sync_copy(zbuf, out_hbm.at[c, pl.ds(s * rpt + r * zr, zr)])
            return carry

        lax.fori_loop(0, rpt // zr, wstep, 0)

    zeros = jnp.zeros((zr, D), jnp.float32)
    return k(hs, src3, dst3, zeros)


def _tc_first(degp, x, W):
    N, Din = x.shape
    Dout = W.shape[1]

    def body(degp_ref, x_ref, w_ref, dis_ref, hs_ref):
        deg = degp_ref[0][:N, :1] + degp_ref[1][:N, :1] + 1.0
        dis = lax.rsqrt(deg)
        dis_ref[...] = dis
        hs_ref[...] = jnp.dot(x_ref[...], w_ref[...],
                              preferred_element_type=jnp.float32) * dis

    return pl.pallas_call(
        body,
        out_shape=(jax.ShapeDtypeStruct((N, 1), jnp.float32),
                   jax.ShapeDtypeStruct((N, Dout), jnp.float32)),
    )(degp, x, W)


def _tc_mid(acc, hs_prev, dis, b, W):
    N, D = hs_prev.shape
    Dout = W.shape[1]

    def body(acc_ref, hsp_ref, dis_ref, b_ref, w_ref, out_ref):
        agg = acc_ref[0][:N] + acc_ref[1][:N] + hsp_ref[...]
        h = jnp.maximum(agg * dis_ref[...] + b_ref[...], 0.0)
        out_ref[...] = jnp.dot(h, w_ref[...],
                               preferred_element_type=jnp.float32) * dis_ref[...]

    return pl.pallas_call(
        body,
        out_shape=jax.ShapeDtypeStruct((N, Dout), jnp.float32),
    )(acc, hs_prev, dis, b.reshape(1, D), W)


def _tc_final(acc, hs_prev, dis, b, Dout):
    N, D = hs_prev.shape

    def body(acc_ref, hsp_ref, dis_ref, b_ref, out_ref):
        agg = acc_ref[0][:N] + acc_ref[1][:N] + hsp_ref[...]
        out_ref[...] = (agg * dis_ref[...])[:, :Dout] + b_ref[...]

    return pl.pallas_call(
        body,
        out_shape=jax.ShapeDtypeStruct((N, Dout), jnp.float32),
    )(acc, hs_prev, dis, b.reshape(1, Dout))


def kernel(x, edge_index, W1, b1, W2, b2, W3, b3):
    N = x.shape[0]
    E = edge_index.shape[1]
    ept = E // NW
    eptp = (-(-ept // (2 * K))) * 2 * K
    nch = eptp // K
    npad = _pad_rows(N) * NS
    e = edge_index.astype(jnp.int32)
    src3 = jnp.pad(e[0].reshape(NW, ept), ((0, 0), (0, eptp - ept)),
                   constant_values=0).reshape(NW, nch, K)
    dst3 = jnp.pad(e[1].reshape(NW, ept), ((0, 0), (0, eptp - ept)),
                   constant_values=npad - 1).reshape(NW, nch, K)
    n_classes = W3.shape[1]
    W3p = jnp.pad(W3, ((0, 0), (0, 128 - n_classes)))

    degp = _sc_degree(dst3, N)
    dis, hs1 = _tc_first(degp, x, W1)
    acc1 = _sc_scatter(hs1, src3, dst3)
    hs2 = _tc_mid(acc1, hs1, dis, b1, W2)
    acc2 = _sc_scatter(hs2, src3, dst3)
    hs3 = _tc_mid(acc2, hs2, dis, b2, W3p)
    acc3 = _sc_scatter(hs3, src3, dst3)
    return _tc_final(acc3, hs3, dis, b3, n_classes)


# speedup vs baseline: 9.0232x; 1.0367x over previous
import functools

import jax
import jax.numpy as jnp
from jax import lax
from jax.experimental import pallas as pl
from jax.experimental.pallas import tpu as pltpu
from jax.experimental.pallas import tpu_sc as plsc

NC = 2
NS = 16
NW = NC * NS
K = 128
H = K // 2


def _pad_rows(N):
    return (-(-(N // NS + 1) // 8)) * 8


def _sc_degree(dst3, zeros, N):
    nch = dst3.shape[1]
    mesh = plsc.VectorSubcoreMesh(
        core_axis_name="c", subcore_axis_name="s", num_cores=NC, num_subcores=NS)
    rpt = _pad_rows(N)
    npad = rpt * NS

    @functools.partial(
        pl.kernel,
        out_type=jax.ShapeDtypeStruct((NC, npad, 128), jnp.float32),
        mesh=mesh,
        scratch_types=[
            pltpu.VMEM((nch, K), jnp.int32),
            pltpu.VMEM((K, 128), jnp.float32),
            pltpu.VMEM_SHARED((npad, 128), jnp.float32),
        ],
    )
    def k(dst_hbm, ones_hbm, zeros_hbm, out_hbm, dst_v, ones_v, acc_sh):
        c = lax.axis_index("c")
        s = lax.axis_index("s")
        w = s * NC + c
        pltpu.sync_copy(dst_hbm.at[w], dst_v)
        pltpu.sync_copy(ones_hbm, ones_v)
        pltpu.sync_copy(zeros_hbm, acc_sh.at[pl.ds(s * rpt, rpt)])
        plsc.subcore_barrier()

        def step(j, carry):
            pltpu.sync_copy(ones_v, acc_sh.at[dst_v.at[j]], add=True)
            return carry

        lax.fori_loop(0, nch, step, 0)
        plsc.subcore_barrier()
        pltpu.sync_copy(acc_sh.at[pl.ds(s * rpt, rpt)],
                        out_hbm.at[c, pl.ds(s * rpt, rpt)])

    ones = jnp.ones((K, 128), jnp.float32)
    return k(dst3, ones, zeros)


def _sc_scatter(hs, src3, dst3, zeros):
    N, D = hs.shape
    nch = src3.shape[1]
    half = nch // 2
    rpt = _pad_rows(N)
    npad = rpt * NS
    mesh = plsc.VectorSubcoreMesh(
        core_axis_name="c", subcore_axis_name="s", num_cores=NC, num_subcores=NS)

    @functools.partial(
        pl.kernel,
        out_type=jax.ShapeDtypeStruct((NC, npad, D), jnp.float32),
        mesh=mesh,
        scratch_types=[
            pltpu.VMEM((half, K), jnp.int32),
            pltpu.VMEM((half, K), jnp.int32),
            pltpu.VMEM((K, D), jnp.float32),
            pltpu.VMEM((K, D), jnp.float32),
            pltpu.VMEM_SHARED((npad, D), jnp.float32),
            pltpu.SemaphoreType.DMA,
            pltpu.SemaphoreType.DMA,
            pltpu.SemaphoreType.DMA,
            pltpu.SemaphoreType.DMA,
        ],
    )
    def k(hs_hbm, src_hbm, dst_hbm, zeros_hbm, out_hbm,
          src_v, dst_v, rows_a, rows_b, acc_sh, ga0, ga1, gb0, gb1):
        c = lax.axis_index("c")
        s = lax.axis_index("s")
        w = s * NC + c
        pltpu.sync_copy(zeros_hbm, acc_sh.at[pl.ds(s * rpt, rpt)])
        plsc.subcore_barrier()

        def start(j, rows, s0, s1):
            pltpu.async_copy(
                hs_hbm.at[src_v.at[j, pl.ds(0, H)]], rows.at[pl.ds(0, H)], s0)
            pltpu.async_copy(
                hs_hbm.at[src_v.at[j, pl.ds(H, H)]], rows.at[pl.ds(H, H)], s1)

        def wait(j, rows, s0, s1):
            pltpu.make_async_copy(
                hs_hbm.at[src_v.at[j, pl.ds(0, H)]], rows.at[pl.ds(0, H)], s0).wait()
            pltpu.make_async_copy(
                hs_hbm.at[src_v.at[j, pl.ds(H, H)]], rows.at[pl.ds(H, H)], s1).wait()

        for h in range(2):
            pltpu.sync_copy(src_hbm.at[w, pl.ds(h * half, half)], src_v)
            pltpu.sync_copy(dst_hbm.at[w, pl.ds(h * half, half)], dst_v)
            start(0, rows_a, ga0, ga1)
            start(1, rows_b, gb0, gb1)

            def step(g, carry):
                j = 2 * g
                wait(j, rows_a, ga0, ga1)
                pltpu.sync_copy(rows_a, acc_sh.at[dst_v.at[j]], add=True)
                start(j + 2, rows_a, ga0, ga1)
                wait(j + 1, rows_b, gb0, gb1)
                pltpu.sync_copy(rows_b, acc_sh.at[dst_v.at[j + 1]], add=True)
                start(j + 3, rows_b, gb0, gb1)
                return carry

            lax.fori_loop(0, half // 2 - 1, step, 0)
            wait(half - 2, rows_a, ga0, ga1)
            pltpu.sync_copy(rows_a, acc_sh.at[dst_v.at[half - 2]], add=True)
            wait(half - 1, rows_b, gb0, gb1)
            pltpu.sync_copy(rows_b, acc_sh.at[dst_v.at[half - 1]], add=True)

        plsc.subcore_barrier()
        pltpu.sync_copy(acc_sh.at[pl.ds(s * rpt, rpt)],
                        out_hbm.at[c, pl.ds(s * rpt, rpt)])

    return k(hs, src3, dst3, zeros)


def _tc_first(degp, x, W):
    N, Din = x.shape
    Dout = W.shape[1]

    def body(degp_ref, x_ref, w_ref, dis_ref, hs_ref):
        deg = degp_ref[0][:N, :1] + degp_ref[1][:N, :1] + 1.0
        dis = lax.rsqrt(deg)
        dis_ref[...] = dis
        hs_ref[...] = jnp.dot(x_ref[...], w_ref[...],
                              preferred_element_type=jnp.float32) * dis

    return pl.pallas_call(
        body,
        out_shape=(jax.ShapeDtypeStruct((N, 1), jnp.float32),
                   jax.ShapeDtypeStruct((N, Dout), jnp.float32)),
    )(degp, x, W)


def _tc_mid(acc, hs_prev, dis, b, W):
    N, D = hs_prev.shape
    Dout = W.shape[1]

    def body(acc_ref, hsp_ref, dis_ref, b_ref, w_ref, out_ref):
        agg = acc_ref[0][:N] + acc_ref[1][:N] + hsp_ref[...]
        h = jnp.maximum(agg * dis_ref[...] + b_ref[...], 0.0)
        out_ref[...] = jnp.dot(h, w_ref[...],
                               preferred_element_type=jnp.float32) * dis_ref[...]

    return pl.pallas_call(
        body,
        out_shape=jax.ShapeDtypeStruct((N, Dout), jnp.float32),
    )(acc, hs_prev, dis, b.reshape(1, D), W)


def _tc_final(acc, hs_prev, dis, b, Dout):
    N, D = hs_prev.shape

    def body(acc_ref, hsp_ref, dis_ref, b_ref, out_ref):
        agg = acc_ref[0][:N] + acc_ref[1][:N] + hsp_ref[...]
        out_ref[...] = (agg * dis_ref[...])[:, :Dout] + b_ref[...]

    return pl.pallas_call(
        body,
        out_shape=jax.ShapeDtypeStruct((N, Dout), jnp.float32),
    )(acc, hs_prev, dis, b.reshape(1, Dout))


def kernel(x, edge_index, W1, b1, W2, b2, W3, b3):
    N = x.shape[0]
    E = edge_index.shape[1]
    ept = E // NW
    eptp = (-(-ept // (2 * K))) * 2 * K
    nch = eptp // K
    rpt = _pad_rows(N)
    npad = rpt * NS
    e = edge_index.astype(jnp.int32)
    src3 = jnp.pad(e[0].reshape(NW, ept), ((0, 0), (0, eptp - ept)),
                   constant_values=0).reshape(NW, nch, K)
    dst3 = jnp.pad(e[1].reshape(NW, ept), ((0, 0), (0, eptp - ept)),
                   constant_values=npad - 1).reshape(NW, nch, K)
    n_classes = W3.shape[1]
    W3p = jnp.pad(W3, ((0, 0), (0, 128 - n_classes)))
    zeros = jnp.zeros((rpt, 128), jnp.float32)

    degp = _sc_degree(dst3, zeros, N)
    dis, hs1 = _tc_first(degp, x, W1)
    acc1 = _sc_scatter(hs1, src3, dst3, zeros)
    hs2 = _tc_mid(acc1, hs1, dis, b1, W2)
    acc2 = _sc_scatter(hs2, src3, dst3, zeros)
    hs3 = _tc_mid(acc2, hs2, dis, b2, W3p)
    acc3 = _sc_scatter(hs3, src3, dst3, zeros)
    return _tc_final(acc3, hs3, dis, b3, n_classes)
